# Initial kernel scaffold; baseline (speedup 1.0000x reference)
import jax
import jax.numpy as jnp
from jax.experimental import pallas as pl


def kernel(logits):
    raise NotImplementedError("write your pallas kernel here")


# speedup vs baseline: 1.0221x; 1.0221x over previous
import functools

import numpy as np
import jax
import jax.numpy as jnp
from jax.experimental import pallas as pl
from jax.experimental.pallas import tpu as pltpu

BLOCK_K = 4096
NEG_INF = np.float32(-np.inf)
TINY = np.float32(np.finfo(np.float32).tiny)

_KS0 = np.uint32(0)
_KS1 = np.uint32(42)
_KS2 = np.uint32(0 ^ 42 ^ 0x1BD11BDA)
_KS = (_KS0, _KS1, _KS2)
_ROT_A = (13, 15, 26, 6)
_ROT_B = (17, 29, 16, 24)


def _threefry_bits(lo):
    x0 = jnp.zeros_like(lo) + _KS0
    x1 = lo + _KS1
    for i, rots in enumerate((_ROT_A, _ROT_B, _ROT_A, _ROT_B, _ROT_A)):
        for r in rots:
            x0 = x0 + x1
            x1 = (x1 << np.uint32(r)) | (x1 >> np.uint32(32 - r))
            x1 = x1 ^ x0
        x0 = x0 + _KS[(i + 1) % 3]
        x1 = x1 + _KS[(i + 2) % 3] + np.uint32(i + 1)
    return x0 ^ x1


def _gumbel_from_bits(bits):
    f = jax.lax.bitcast_convert_type(
        (bits >> np.uint32(9)) | np.uint32(0x3F800000), jnp.float32
    ) - np.float32(1.0)
    u = jnp.maximum(TINY, f * (np.float32(1.0) - TINY) + TINY)
    return -jnp.log(-jnp.log(u))


def _fused_kernel(n_cols, n_blocks, x_ref, samp_ref, logp_ref, ent_ref,
                  m_ref, s_ref, t_ref, gm_ref, gi_ref, gx_ref):
    j = pl.program_id(0)
    rows = x_ref.shape[0]
    bk = x_ref.shape[1]

    @pl.when(j == 0)
    def _init():
        m_ref[...] = jnp.full((rows, 1), NEG_INF, jnp.float32)
        s_ref[...] = jnp.zeros((rows, 1), jnp.float32)
        t_ref[...] = jnp.zeros((rows, 1), jnp.float32)
        gm_ref[...] = jnp.full((rows, 1), NEG_INF, jnp.float32)
        gi_ref[...] = jnp.zeros((rows, 1), jnp.int32)
        gx_ref[...] = jnp.zeros((rows, 1), jnp.float32)

    x = x_ref[...]
    col = j * bk + jax.lax.broadcasted_iota(jnp.int32, (rows, bk), 1)
    valid = col < n_cols

    row = jax.lax.broadcasted_iota(jnp.int32, (rows, bk), 0)
    flat = (row * n_cols + col).astype(jnp.uint32)
    g = _gumbel_from_bits(_threefry_bits(flat))

    xm = jnp.where(valid, x, NEG_INF)
    m_old = m_ref[...]
    m_new = jnp.maximum(m_old, jnp.max(xm, axis=1, keepdims=True))
    e = jnp.where(valid, jnp.exp(x - m_new), np.float32(0.0))
    te = jnp.where(valid, (x - m_new) * e, np.float32(0.0))
    scale = jnp.exp(m_old - m_new)
    carry = jnp.where(
        scale > 0,
        (t_ref[...] + (m_old - m_new) * s_ref[...]) * scale,
        np.float32(0.0),
    )
    s_ref[...] = s_ref[...] * scale + jnp.sum(e, axis=1, keepdims=True)
    t_ref[...] = carry + jnp.sum(te, axis=1, keepdims=True)
    m_ref[...] = m_new

    z = jnp.where(valid, x + g, NEG_INF)
    zmax = jnp.max(z, axis=1, keepdims=True)
    idx = jnp.min(
        jnp.where(z == zmax, col, jnp.int32(np.iinfo(np.int32).max)),
        axis=1, keepdims=True,
    )
    x_at = jnp.max(jnp.where(col == idx, x, NEG_INF), axis=1, keepdims=True)
    better = zmax > gm_ref[...]
    gm_ref[...] = jnp.where(better, zmax, gm_ref[...])
    gi_ref[...] = jnp.where(better, idx, gi_ref[...])
    gx_ref[...] = jnp.where(better, x_at, gx_ref[...])

    @pl.when(j == n_blocks - 1)
    def _finish():
        s = s_ref[...]
        logs = jnp.log(s)
        lse = m_ref[...] + logs
        samp_ref[...] = gi_ref[...]
        logp_ref[...] = gx_ref[...] - lse
        ent_ref[...] = logs - t_ref[...] / s


def kernel(logits):
    rows, n_cols = logits.shape
    n_blocks = pl.cdiv(n_cols, BLOCK_K)
    out_shapes = (
        jax.ShapeDtypeStruct((rows, 1), jnp.int32),
        jax.ShapeDtypeStruct((rows, 1), jnp.float32),
        jax.ShapeDtypeStruct((rows, 1), jnp.float32),
    )
    samp, sel_logp, entropy = pl.pallas_call(
        functools.partial(_fused_kernel, n_cols, n_blocks),
        grid=(n_blocks,),
        in_specs=[pl.BlockSpec((rows, BLOCK_K), lambda j: (0, j))],
        out_specs=(
            pl.BlockSpec((rows, 1), lambda j: (0, 0)),
            pl.BlockSpec((rows, 1), lambda j: (0, 0)),
            pl.BlockSpec((rows, 1), lambda j: (0, 0)),
        ),
        out_shape=out_shapes,
        scratch_shapes=[
            pltpu.VMEM((rows, 1), jnp.float32),
            pltpu.VMEM((rows, 1), jnp.float32),
            pltpu.VMEM((rows, 1), jnp.float32),
            pltpu.VMEM((rows, 1), jnp.float32),
            pltpu.VMEM((rows, 1), jnp.int32),
            pltpu.VMEM((rows, 1), jnp.float32),
        ],
    )(logits)
    return samp[:, 0], sel_logp[:, 0], entropy[:, 0]
